# baseline (speedup 1.0000x reference)
import functools

import jax
import jax.numpy as jnp
from jax import lax
from jax.experimental import pallas as pl
from jax.experimental.pallas import tpu as pltpu
from jax.experimental.pallas import tpu_sc as plsc

BATCH = 16384
FACTORS = 64
NUM_CORES = 2
NUM_SUBCORES = 16
NUM_WORKERS = NUM_CORES * NUM_SUBCORES
BPW = BATCH // NUM_WORKERS
IDX_CHUNK = 128
NCHUNK = BPW // IDX_CHUNK
UNROLL = 16

_mesh = plsc.VectorSubcoreMesh(core_axis_name="c", subcore_axis_name="s")


@functools.partial(
    pl.kernel,
    out_type=jax.ShapeDtypeStruct((BATCH,), jnp.float32),
    mesh=_mesh,
    compiler_params=pltpu.CompilerParams(use_tc_tiling_on_sc=False),
    scratch_types=[
        pltpu.VMEM((NCHUNK, IDX_CHUNK), jnp.int32),
        pltpu.VMEM((NCHUNK, IDX_CHUNK), jnp.int32),
        pltpu.VMEM((BPW, FACTORS), jnp.float32),
        pltpu.VMEM((BPW, FACTORS), jnp.float32),
        pltpu.VMEM((BPW,), jnp.float32),
        pltpu.SemaphoreType.DMA,
        pltpu.SemaphoreType.DMA,
    ],
)
def _bpr_dot(users_hbm, item_hbm, utab_hbm, itab_hbm, out_hbm,
             uidx_v, iidx_v, urows_v, irows_v, out_v, sem_idx, sem_rows):
    wid = lax.axis_index("s") * NUM_CORES + lax.axis_index("c")
    base = wid * BPW

    for k in range(NCHUNK):
        pltpu.async_copy(users_hbm.at[pl.ds(base + k * IDX_CHUNK, IDX_CHUNK)],
                         uidx_v.at[k], sem_idx)
        pltpu.async_copy(item_hbm.at[pl.ds(base + k * IDX_CHUNK, IDX_CHUNK)],
                         iidx_v.at[k], sem_idx)
    for k in range(NCHUNK):
        pltpu.make_async_copy(users_hbm.at[pl.ds(base, IDX_CHUNK)],
                              uidx_v.at[k], sem_idx).wait()
        pltpu.make_async_copy(item_hbm.at[pl.ds(base, IDX_CHUNK)],
                              iidx_v.at[k], sem_idx).wait()

    for k in range(NCHUNK):
        pltpu.async_copy(utab_hbm.at[uidx_v.at[k]],
                         urows_v.at[pl.ds(k * IDX_CHUNK, IDX_CHUNK)], sem_rows)
        pltpu.async_copy(itab_hbm.at[iidx_v.at[k]],
                         irows_v.at[pl.ds(k * IDX_CHUNK, IDX_CHUNK)], sem_rows)
    for k in range(NCHUNK):
        pltpu.make_async_copy(utab_hbm.at[uidx_v.at[k]],
                              urows_v.at[pl.ds(k * IDX_CHUNK, IDX_CHUNK)],
                              sem_rows).wait()
        pltpu.make_async_copy(itab_hbm.at[iidx_v.at[k]],
                              irows_v.at[pl.ds(k * IDX_CHUNK, IDX_CHUNK)],
                              sem_rows).wait()

    lane = lax.iota(jnp.int32, 16)

    def _hsum(v):
        for s in (8, 4, 2, 1):
            v = v + jnp.take_along_axis(v, lane ^ s, axis=0,
                                        mode="promise_in_bounds")
        return v

    def body(g, carry):
        out16 = jnp.zeros((16,), jnp.float32)
        for j in range(UNROLL):
            r = g * UNROLL + j
            acc = urows_v[r, pl.ds(0, 16)] * irows_v[r, pl.ds(0, 16)]
            for c in range(1, FACTORS // 16):
                acc = acc + (urows_v[r, pl.ds(c * 16, 16)]
                             * irows_v[r, pl.ds(c * 16, 16)])
            out16 = jnp.where(lane == j, _hsum(acc), out16)
        out_v[pl.ds(g * UNROLL, UNROLL)] = out16
        return carry

    lax.fori_loop(0, BPW // UNROLL, body, 0, unroll=False)

    pltpu.sync_copy(out_v, out_hbm.at[pl.ds(base, BPW)])


def kernel(users, item, user_emb_weight, item_emb_weight):
    return _bpr_dot(users.astype(jnp.int32), item.astype(jnp.int32),
                    user_emb_weight, item_emb_weight)


# speedup vs baseline: 1.4716x; 1.4716x over previous
import functools

import jax
import jax.numpy as jnp
from jax import lax
from jax.experimental import pallas as pl
from jax.experimental.pallas import tpu as pltpu
from jax.experimental.pallas import tpu_sc as plsc

BATCH = 16384
FACTORS = 64
NUM_CORES = 2
NUM_SUBCORES = 16
NUM_WORKERS = NUM_CORES * NUM_SUBCORES
BPW = BATCH // NUM_WORKERS
IDX_CHUNK = 128
NCHUNK = BPW // IDX_CHUNK
ROWS_PER_CHUNK = 32
NCHUNKS = BPW // ROWS_PER_CHUNK

_mesh = plsc.VectorSubcoreMesh(core_axis_name="c", subcore_axis_name="s")


@functools.partial(
    pl.kernel,
    out_type=jax.ShapeDtypeStruct((BATCH,), jnp.float32),
    mesh=_mesh,
    scratch_types=[
        pltpu.VMEM((NCHUNK, IDX_CHUNK), jnp.int32),
        pltpu.VMEM((NCHUNK, IDX_CHUNK), jnp.int32),
        pltpu.VMEM((ROWS_PER_CHUNK, 8, FACTORS), jnp.float32),
        pltpu.VMEM((ROWS_PER_CHUNK, 8, FACTORS), jnp.float32),
        pltpu.VMEM((BPW,), jnp.float32),
        pltpu.SemaphoreType.DMA,
        pltpu.SemaphoreType.DMA,
    ],
)
def _bpr_dot(users_hbm, item_hbm, utab_hbm, itab_hbm, out_hbm,
             uidx_v, iidx_v, utile_v, itile_v, out_v, sem_idx, sem_rows):
    wid = lax.axis_index("s") * NUM_CORES + lax.axis_index("c")
    base = wid * BPW

    for k in range(NCHUNK):
        pltpu.async_copy(users_hbm.at[pl.ds(base + k * IDX_CHUNK, IDX_CHUNK)],
                         uidx_v.at[k], sem_idx)
        pltpu.async_copy(item_hbm.at[pl.ds(base + k * IDX_CHUNK, IDX_CHUNK)],
                         iidx_v.at[k], sem_idx)
    for k in range(NCHUNK):
        pltpu.make_async_copy(users_hbm.at[pl.ds(base, IDX_CHUNK)],
                              uidx_v.at[k], sem_idx).wait()
        pltpu.make_async_copy(item_hbm.at[pl.ds(base, IDX_CHUNK)],
                              iidx_v.at[k], sem_idx).wait()

    lane = lax.iota(jnp.int32, 16)

    def _hsum(v):
        for s in (8, 4, 2, 1):
            v = v + jnp.take_along_axis(v, lane ^ s, axis=0,
                                        mode="promise_in_bounds")
        return v

    def _idx16(idx_ref, a):
        return idx_ref[a // 8, pl.ds((a % 8) * 16, 16)]

    def chunk_body(c, carry):
        def fire(k, carry2):
            a = c * (ROWS_PER_CHUNK // 16) + k
            ubase16 = _idx16(uidx_v, a) & ~7
            ibase16 = _idx16(iidx_v, a) & ~7
            for j in range(16):
                r = k * 16 + j
                ub = pl.multiple_of(ubase16[j], 8)
                ib = pl.multiple_of(ibase16[j], 8)
                pltpu.async_copy(utab_hbm.at[pl.ds(ub, 8)],
                                 utile_v.at[r], sem_rows)
                pltpu.async_copy(itab_hbm.at[pl.ds(ib, 8)],
                                 itile_v.at[r], sem_rows)
            return carry2

        lax.fori_loop(0, ROWS_PER_CHUNK // 16, fire, 0, unroll=False)
        pltpu.make_async_copy(utab_hbm.at[pl.ds(0, 8 * ROWS_PER_CHUNK)],
                              utile_v, sem_rows).wait()
        pltpu.make_async_copy(itab_hbm.at[pl.ds(0, 8 * ROWS_PER_CHUNK)],
                              itile_v, sem_rows).wait()

        def compute(k, carry2):
            a = c * (ROWS_PER_CHUNK // 16) + k
            usub16 = _idx16(uidx_v, a) & 7
            isub16 = _idx16(iidx_v, a) & 7
            out16 = jnp.zeros((16,), jnp.float32)
            for j in range(16):
                r = k * 16 + j
                us = usub16[j]
                isx = isub16[j]
                acc = (utile_v[r, us, pl.ds(0, 16)]
                       * itile_v[r, isx, pl.ds(0, 16)])
                for cc in range(1, FACTORS // 16):
                    acc = acc + (utile_v[r, us, pl.ds(cc * 16, 16)]
                                 * itile_v[r, isx, pl.ds(cc * 16, 16)])
                out16 = jnp.where(lane == j, _hsum(acc), out16)
            out_v[pl.ds(a * 16, 16)] = out16
            return carry2

        lax.fori_loop(0, ROWS_PER_CHUNK // 16, compute, 0, unroll=False)
        return carry

    lax.fori_loop(0, NCHUNKS, chunk_body, 0, unroll=False)

    pltpu.sync_copy(out_v, out_hbm.at[pl.ds(base, BPW)])


def kernel(users, item, user_emb_weight, item_emb_weight):
    return _bpr_dot(users.astype(jnp.int32), item.astype(jnp.int32),
                    user_emb_weight, item_emb_weight)


# speedup vs baseline: 2.3330x; 1.5853x over previous
import functools

import jax
import jax.numpy as jnp
from jax import lax
from jax.experimental import pallas as pl
from jax.experimental.pallas import tpu as pltpu
from jax.experimental.pallas import tpu_sc as plsc

BATCH = 16384
FACTORS = 64
NUM_CORES = 2
NUM_SUBCORES = 16
NUM_WORKERS = NUM_CORES * NUM_SUBCORES
BPW = BATCH // NUM_WORKERS
IDX_CHUNK = 128
NCHUNK = BPW // IDX_CHUNK
NGROUPS = BPW // 16
NSUB = 8

_mesh = plsc.VectorSubcoreMesh(core_axis_name="c", subcore_axis_name="s")


@functools.partial(
    pl.kernel,
    out_type=jax.ShapeDtypeStruct((BATCH,), jnp.float32),
    mesh=_mesh,
    compiler_params=pltpu.CompilerParams(needs_layout_passes=False),
    scratch_types=[
        pltpu.VMEM((NCHUNK, IDX_CHUNK), jnp.int32),
        pltpu.VMEM((NCHUNK, IDX_CHUNK), jnp.int32),
        pltpu.VMEM((2, 2, FACTORS, 128), jnp.float32),
        pltpu.VMEM((2, 2, FACTORS, 128), jnp.float32),
        pltpu.VMEM((BPW,), jnp.float32),
        pltpu.SemaphoreType.DMA,
        pltpu.SemaphoreType.DMA,
        pltpu.SemaphoreType.DMA,
    ],
)
def _bpr_dot(users_hbm, item_hbm, utab_hbm, itab_hbm, out_hbm,
             uidx_v, iidx_v, uwin_v, iwin_v, out_v, sem_idx, sem_a, sem_b):
    wid = lax.axis_index("s") * NUM_CORES + lax.axis_index("c")
    base = wid * BPW

    for k in range(NCHUNK):
        pltpu.async_copy(users_hbm.at[pl.ds(base + k * IDX_CHUNK, IDX_CHUNK)],
                         uidx_v.at[k], sem_idx)
        pltpu.async_copy(item_hbm.at[pl.ds(base + k * IDX_CHUNK, IDX_CHUNK)],
                         iidx_v.at[k], sem_idx)
    for k in range(NCHUNK):
        pltpu.make_async_copy(users_hbm.at[pl.ds(base, IDX_CHUNK)],
                              uidx_v.at[k], sem_idx).wait()
        pltpu.make_async_copy(item_hbm.at[pl.ds(base, IDX_CHUNK)],
                              iidx_v.at[k], sem_idx).wait()

    lane = lax.iota(jnp.int32, 16)
    sems = (sem_a, sem_b)

    def _hsum(v):
        for s in (8, 4, 2, 1):
            v = v + jnp.take_along_axis(v, lane ^ s, axis=0,
                                        mode="promise_in_bounds")
        return v

    def group_body(a, carry):
        uidx16 = uidx_v[a // 8, pl.ds((a % 8) * 16, 16)]
        iidx16 = iidx_v[a // 8, pl.ds((a % 8) * 16, 16)]
        ucol = (uidx16 >> 7) << 7
        icol = (iidx16 >> 7) << 7
        uoff = uidx16 & 127
        ioff = iidx16 & 127

        def fire(sub):
            slot = sub % 2
            for jj in range(2):
                j = sub * 2 + jj
                uc = pl.multiple_of(ucol[j], 128)
                ic = pl.multiple_of(icol[j], 128)
                pltpu.async_copy(utab_hbm.at[:, pl.ds(uc, 128)],
                                 uwin_v.at[slot, jj], sems[slot])
                pltpu.async_copy(itab_hbm.at[:, pl.ds(ic, 128)],
                                 iwin_v.at[slot, jj], sems[slot])

        def wait(sub):
            slot = sub % 2
            for jj in range(2):
                pltpu.make_async_copy(utab_hbm.at[:, pl.ds(0, 128)],
                                      uwin_v.at[slot, jj], sems[slot]).wait()
                pltpu.make_async_copy(itab_hbm.at[:, pl.ds(0, 128)],
                                      iwin_v.at[slot, jj], sems[slot]).wait()

        out16 = jnp.zeros((16,), jnp.float32)
        fire(0)
        for sub in range(NSUB):
            if sub + 1 < NSUB:
                fire(sub + 1)
            wait(sub)
            slot = sub % 2
            for jj in range(2):
                j = sub * 2 + jj
                ul = jnp.full((16,), uoff[j], jnp.int32)
                il = jnp.full((16,), ioff[j], jnp.int32)
                acc = None
                for c in range(FACTORS // 16):
                    fidx = lane + (c * 16)
                    ug = plsc.load_gather(uwin_v.at[slot, jj], [fidx, ul])
                    ig = plsc.load_gather(iwin_v.at[slot, jj], [fidx, il])
                    p = ug * ig
                    acc = p if acc is None else acc + p
                out16 = jnp.where(lane == j, _hsum(acc), out16)
        out_v[pl.ds(a * 16, 16)] = out16
        return carry

    lax.fori_loop(0, NGROUPS, group_body, 0, unroll=False)

    pltpu.sync_copy(out_v, out_hbm.at[pl.ds(base, BPW)])


def kernel(users, item, user_emb_weight, item_emb_weight):
    return _bpr_dot(users.astype(jnp.int32), item.astype(jnp.int32),
                    user_emb_weight.T, item_emb_weight.T)


# speedup vs baseline: 2.3806x; 1.0204x over previous
import functools

import jax
import jax.numpy as jnp
from jax import lax
from jax.experimental import pallas as pl
from jax.experimental.pallas import tpu as pltpu
from jax.experimental.pallas import tpu_sc as plsc

BATCH = 16384
FACTORS = 64
NUM_CORES = 2
NUM_SUBCORES = 16
NUM_WORKERS = NUM_CORES * NUM_SUBCORES
BPW = BATCH // NUM_WORKERS
IDX_CHUNK = 128
NCHUNK = BPW // IDX_CHUNK
NGROUPS = BPW // 16
NSUB = 8

_mesh = plsc.VectorSubcoreMesh(core_axis_name="c", subcore_axis_name="s")


@functools.partial(
    pl.kernel,
    out_type=jax.ShapeDtypeStruct((BATCH,), jnp.float32),
    mesh=_mesh,
    compiler_params=pltpu.CompilerParams(needs_layout_passes=False),
    scratch_types=[
        pltpu.VMEM((NCHUNK, IDX_CHUNK), jnp.int32),
        pltpu.VMEM((NCHUNK, IDX_CHUNK), jnp.int32),
        pltpu.VMEM((3, 2, FACTORS, 128), jnp.float32),
        pltpu.VMEM((3, 2, FACTORS, 128), jnp.float32),
        pltpu.VMEM((BPW,), jnp.float32),
        pltpu.SemaphoreType.DMA,
        pltpu.SemaphoreType.DMA,
        pltpu.SemaphoreType.DMA,
        pltpu.SemaphoreType.DMA,
    ],
)
def _bpr_dot(users_hbm, item_hbm, utab_hbm, itab_hbm, out_hbm,
             uidx_v, iidx_v, uwin_v, iwin_v, out_v, sem_idx,
             sem_a, sem_b, sem_c):
    wid = lax.axis_index("s") * NUM_CORES + lax.axis_index("c")
    base = wid * BPW

    for k in range(NCHUNK):
        pltpu.async_copy(users_hbm.at[pl.ds(base + k * IDX_CHUNK, IDX_CHUNK)],
                         uidx_v.at[k], sem_idx)
        pltpu.async_copy(item_hbm.at[pl.ds(base + k * IDX_CHUNK, IDX_CHUNK)],
                         iidx_v.at[k], sem_idx)
    for k in range(NCHUNK):
        pltpu.make_async_copy(users_hbm.at[pl.ds(base, IDX_CHUNK)],
                              uidx_v.at[k], sem_idx).wait()
        pltpu.make_async_copy(item_hbm.at[pl.ds(base, IDX_CHUNK)],
                              iidx_v.at[k], sem_idx).wait()

    lane = lax.iota(jnp.int32, 16)
    sems = (sem_a, sem_b, sem_c)

    def _hsum(v):
        for s in (8, 4, 2, 1):
            v = v + jnp.take_along_axis(v, lane ^ s, axis=0,
                                        mode="promise_in_bounds")
        return v

    def group_body(a, carry):
        uidx16 = uidx_v[a // 8, pl.ds((a % 8) * 16, 16)]
        iidx16 = iidx_v[a // 8, pl.ds((a % 8) * 16, 16)]
        ucol = (uidx16 >> 7) << 7
        icol = (iidx16 >> 7) << 7
        uoff = uidx16 & 127
        ioff = iidx16 & 127

        def fire(sub):
            slot = sub % 3
            for jj in range(2):
                j = sub * 2 + jj
                uc = pl.multiple_of(ucol[j], 128)
                ic = pl.multiple_of(icol[j], 128)
                pltpu.async_copy(utab_hbm.at[:, pl.ds(uc, 128)],
                                 uwin_v.at[slot, jj], sems[slot])
                pltpu.async_copy(itab_hbm.at[:, pl.ds(ic, 128)],
                                 iwin_v.at[slot, jj], sems[slot])

        def wait(sub):
            slot = sub % 3
            for jj in range(2):
                pltpu.make_async_copy(utab_hbm.at[:, pl.ds(0, 128)],
                                      uwin_v.at[slot, jj], sems[slot]).wait()
                pltpu.make_async_copy(itab_hbm.at[:, pl.ds(0, 128)],
                                      iwin_v.at[slot, jj], sems[slot]).wait()

        out16 = jnp.zeros((16,), jnp.float32)
        fire(0)
        fire(1)
        for sub in range(NSUB):
            if sub + 2 < NSUB:
                fire(sub + 2)
            wait(sub)
            slot = sub % 3
            for jj in range(2):
                j = sub * 2 + jj
                ul = jnp.full((16,), uoff[j], jnp.int32)
                il = jnp.full((16,), ioff[j], jnp.int32)
                acc = None
                for c in range(FACTORS // 16):
                    fidx = lane + (c * 16)
                    ug = plsc.load_gather(uwin_v.at[slot, jj], [fidx, ul])
                    ig = plsc.load_gather(iwin_v.at[slot, jj], [fidx, il])
                    p = ug * ig
                    acc = p if acc is None else acc + p
                out16 = jnp.where(lane == j, _hsum(acc), out16)
        out_v[pl.ds(a * 16, 16)] = out16
        return carry

    lax.fori_loop(0, NGROUPS, group_body, 0, unroll=False)

    pltpu.sync_copy(out_v, out_hbm.at[pl.ds(base, BPW)])


def kernel(users, item, user_emb_weight, item_emb_weight):
    return _bpr_dot(users.astype(jnp.int32), item.astype(jnp.int32),
                    user_emb_weight.T, item_emb_weight.T)
